# Initial kernel scaffold; baseline (speedup 1.0000x reference)
import jax
import jax.numpy as jnp
from jax.experimental import pallas as pl


def kernel(x, perms):
    raise NotImplementedError("write your pallas kernel here")


# speedup vs baseline: 1.3233x; 1.3233x over previous
import functools

import jax
import jax.numpy as jnp
from jax import lax
from jax.experimental import pallas as pl
from jax.experimental.pallas import tpu as pltpu
from jax.experimental.pallas import tpu_sc as plsc

L = 8192
B = 64

_info = plsc.get_sparse_core_info()
_NC, _NS, _LANES = _info.num_cores, _info.num_subcores, _info.num_lanes
_NW = _NC * _NS
_ROWS_PER_W = B // _NW

_mesh = plsc.VectorSubcoreMesh(core_axis_name="c", subcore_axis_name="s")


@functools.partial(
    pl.kernel,
    mesh=_mesh,
    out_type=jax.ShapeDtypeStruct((B, L), jnp.float32),
    scratch_types=[
        pltpu.VMEM((L,), jnp.float32),
        pltpu.VMEM((L,), jnp.int32),
        pltpu.VMEM((L,), jnp.float32),
    ],
    compiler_params=pltpu.CompilerParams(needs_layout_passes=False),
)
def _interleave(x_hbm, p_hbm, out_hbm, xv, pv, ov):
    wid = lax.axis_index("s") * _NC + lax.axis_index("c")
    for r in range(_ROWS_PER_W):
        row = wid * _ROWS_PER_W + r
        pltpu.sync_copy(x_hbm.at[row], xv)
        pltpu.sync_copy(p_hbm.at[row], pv)

        def body(j, carry):
            idx = pv[pl.ds(j * _LANES, _LANES)]
            ov[pl.ds(j * _LANES, _LANES)] = plsc.load_gather(xv, [idx])
            return carry

        lax.fori_loop(0, L // _LANES, body, 0, unroll=8)
        pltpu.sync_copy(ov, out_hbm.at[row])


def kernel(x, perms):
    out = _interleave(x[:, :, 0], perms)
    return out[:, :, None]
